# Initial kernel scaffold; baseline (speedup 1.0000x reference)
import jax
import jax.numpy as jnp
from jax.experimental import pallas as pl


def kernel(locs):
    raise NotImplementedError("write your pallas kernel here")


# speedup vs baseline: 14.4039x; 14.4039x over previous
import functools

import jax
import jax.numpy as jnp
import numpy as np
from jax import lax
from jax.experimental import pallas as pl
from jax.experimental.pallas import tpu as pltpu
from jax.experimental.pallas import tpu_sc as plsc

_RADIUS = 0.4
_MAXG = 96.0
_MAXC = 128
_B = 2
_N = 4096
_NW = 32
_WPB = _NW // _B
_OWN = _B * _N // _NW
_NCHUNK = 8
_ROWS = _N // _WPB // _NCHUNK
_NGRP = _N // 16

_SIGN = np.int32(-2147483648)
_R2 = np.float32(_RADIUS * _RADIUS)

_PACK = np.zeros((_N, _NGRP), np.float32)
_PACK[np.arange(_N), np.arange(_N) >> 4] = (2.0 ** (np.arange(_N) & 15))


def _cells_body(x_ref, y_ref, z_ref, key_ref):
    r32 = jnp.float32(_RADIUS)
    for b in range(_B):
        arrs = (x_ref[b], y_ref[b], z_ref[b])
        cells = []
        gds = []
        for arr in arrs:
            lo = jnp.min(arr)
            up = jnp.max(arr)
            gdim = jnp.ceil(jnp.clip((up - lo) / r32, 0.0, jnp.float32(_MAXG)))
            center = (lo + up) / 2.0
            low2 = center - gdim * r32 / 2.0
            cell = jnp.floor((arr - low2) / r32)
            cell = jnp.clip(cell, 0.0, gdim - 1.0)
            cells.append(cell.astype(jnp.int32))
            gds.append(gdim.astype(jnp.int32))
        cid = (cells[0] * gds[1] + cells[1]) * gds[2] + cells[2]
        pos = (lax.broadcasted_iota(jnp.int32, arrs[0].shape, 0) * arrs[0].shape[1]
               + lax.broadcasted_iota(jnp.int32, arrs[0].shape, 1))
        key_ref[b] = cid * 4096 + pos


def _compute_keys(xs, ys, zs):
    shape = (_B, _N // 128, 128)
    key = pl.pallas_call(
        _cells_body,
        out_shape=jax.ShapeDtypeStruct(shape, jnp.int32),
    )(xs.reshape(shape), ys.reshape(shape), zs.reshape(shape))
    return key.reshape(_B * _N)


@functools.cache
def _make_rank_scatter():
    mesh = plsc.VectorSubcoreMesh(core_axis_name="c", subcore_axis_name="s")
    return functools.partial(
        pl.kernel,
        out_type=[
            jax.ShapeDtypeStruct((_B * _N,), jnp.int32),
            jax.ShapeDtypeStruct((_B * _N,), jnp.float32),
            jax.ShapeDtypeStruct((_B * _N,), jnp.float32),
            jax.ShapeDtypeStruct((_B * _N,), jnp.float32),
        ],
        mesh=mesh,
        compiler_params=pltpu.CompilerParams(needs_layout_passes=False),
        scratch_types=[
            pltpu.VMEM((_N,), jnp.int32),
            pltpu.VMEM((_N,), jnp.int32),
            pltpu.VMEM((_OWN,), jnp.float32),
            pltpu.VMEM((_OWN,), jnp.float32),
            pltpu.VMEM((_OWN,), jnp.float32),
            pltpu.VMEM((2, 128), jnp.int32),
            pltpu.SemaphoreType.DMA,
        ],
    )(_rank_scatter_body)


def _rank_scatter_body(key_hbm, x_hbm, y_hbm, z_hbm,
                       key_s_hbm, xs_hbm, ys_hbm, zs_hbm,
                       keys_v, keysx_v, xo_v, yo_v, zo_v, idx_v, sem):
    wid = lax.axis_index("s") * 2 + lax.axis_index("c")
    b = wid // _WPB
    bbase = b * _N
    obase = (wid % _WPB) * _OWN

    pltpu.sync_copy(key_hbm.at[pl.ds(bbase, _N)], keys_v)
    pltpu.sync_copy(x_hbm.at[pl.ds(bbase + obase, _OWN)], xo_v)
    pltpu.sync_copy(y_hbm.at[pl.ds(bbase + obase, _OWN)], yo_v)
    pltpu.sync_copy(z_hbm.at[pl.ds(bbase + obase, _OWN)], zo_v)

    def flip(c, _):
        keysx_v[pl.ds(c * 16, 16)] = keys_v[pl.ds(c * 16, 16)] ^ _SIGN
        return 0

    lax.fori_loop(0, _N // 16, flip, 0)

    iota = lax.iota(jnp.int32, 16)

    def per_group(c0, _):
        kvec = keysx_v[pl.ds(obase + c0 * 16, 16)]

        def per_chunk(c, accs):
            kv = keysx_v[pl.ds(c * 16, 16)]
            return tuple(accs[l] + jnp.where(kv < kvec[l], 1, 0)
                         for l in range(16))

        accs = lax.fori_loop(
            0, _N // 16, per_chunk,
            tuple(jnp.zeros((16,), jnp.int32) for _ in range(16)))
        rankv = jnp.zeros((16,), jnp.int32)
        for l in range(16):
            rankv = jnp.where(iota == l, bbase + jnp.sum(accs[l]), rankv)
        idx_v[c0 // 8, pl.ds((c0 % 8) * 16, 16)] = rankv
        return 0

    lax.fori_loop(0, _OWN // 16, per_group, 0)

    for c in range(2):
        sl = pl.ds(c * 128, 128)
        pltpu.async_copy(keys_v.at[pl.ds(obase + c * 128, 128)],
                         key_s_hbm.at[idx_v.at[c]], sem).wait()
        pltpu.async_copy(xo_v.at[sl], xs_hbm.at[idx_v.at[c]], sem).wait()
        pltpu.async_copy(yo_v.at[sl], ys_hbm.at[idx_v.at[c]], sem).wait()
        pltpu.async_copy(zo_v.at[sl], zs_hbm.at[idx_v.at[c]], sem).wait()


def _mask_body(a_ref, bT_ref, p_ref, bits_ref):
    a = a_ref[0]
    bT = bT_ref[0]
    x_i, y_i, z_i = a[:, 0:1], a[:, 1:2], a[:, 2:3]
    sq_i = (x_i * x_i + y_i * y_i) + z_i * z_i
    bx, by, bz = bT[0:1, :], bT[1:2, :], bT[2:3, :]
    sq_j = (bx * bx + by * by) + bz * bz
    dot = lax.dot_general(a, bT, (((1,), (0,)), ((), ())),
                          precision=lax.Precision.DEFAULT)
    d2 = sq_i + sq_j - 2.0 * dot
    maskf = jnp.where(d2 <= _R2, 1.0, 0.0)
    pack = lax.dot_general(maskf, p_ref[...], (((1,), (0,)), ((), ())),
                           precision=lax.Precision.DEFAULT)
    bits_ref[0] = pack.astype(jnp.int32)


def _compute_bits(locs_s, locs_sT):
    return pl.pallas_call(
        _mask_body,
        grid=(_B, _N // 128),
        in_specs=[
            pl.BlockSpec((1, 128, 3), lambda b, r: (b, r, 0)),
            pl.BlockSpec((1, 3, _N), lambda b, r: (b, 0, 0)),
            pl.BlockSpec((_N, _NGRP), lambda b, r: (0, 0)),
        ],
        out_specs=pl.BlockSpec((1, 128, _NGRP), lambda b, r: (b, r, 0)),
        out_shape=jax.ShapeDtypeStruct((_B, _N, _NGRP), jnp.int32),
    )(locs_s, locs_sT, jnp.asarray(_PACK))


@functools.cache
def _make_emit():
    mesh = plsc.VectorSubcoreMesh(core_axis_name="c", subcore_axis_name="s")
    return functools.partial(
        pl.kernel,
        out_type=[
            jax.ShapeDtypeStruct((_B * _N,), jnp.float32),
            jax.ShapeDtypeStruct((_B * _N * _MAXC,), jnp.float32),
        ],
        mesh=mesh,
        compiler_params=pltpu.CompilerParams(needs_layout_passes=False),
        scratch_types=[
            pltpu.VMEM((_N,), jnp.int32),
            pltpu.VMEM((_N,), jnp.float32),
            pltpu.VMEM((_ROWS, _NGRP), jnp.int32),
            pltpu.VMEM((_ROWS * _MAXC + 16,), jnp.float32),
            pltpu.SemaphoreType.DMA,
        ],
    )(_emit_body)


def _emit_body(key_s_hbm, bits_hbm, idxs_hbm, nbr_hbm,
               keys_v, ordf_v, rows_v, out_v, sem):
    wid = lax.axis_index("s") * 2 + lax.axis_index("c")
    b = wid // _WPB
    ws = wid % _WPB
    bbase = b * _N

    pltpu.sync_copy(key_s_hbm.at[pl.ds(bbase, _N)], keys_v)

    iota = lax.iota(jnp.int32, 16)

    def unpack(c, _):
        sl = pl.ds(c * 16, 16)
        ordf_v[sl] = (keys_v[sl] & 4095).astype(jnp.float32)
        return 0

    lax.fori_loop(0, _N // 16, unpack, 0)

    def per_chunk(c, _):
        row0 = (c * _WPB + ws) * _ROWS
        cp = pltpu.async_copy(bits_hbm.at[pl.ds(bbase + row0, _ROWS)],
                              rows_v, sem)

        def fill(k, _):
            out_v[pl.ds(k * 16, 16)] = jnp.full((16,), -1.0, jnp.float32)
            return 0

        lax.fori_loop(0, (_ROWS * _MAXC + 16) // 16, fill, 0)
        cp.wait()

        def per_row(p, _):
            def per_wchunk(cb, cnt):
                words = rows_v[p, pl.ds(cb * 16, 16)]
                nz = words != 0

                def cond(carry):
                    nz, cnt = carry
                    return jnp.any(nz) & (cnt < _MAXC)

                def body(carry):
                    nz, cnt = carry
                    l = plsc.all_reduce_ffs(nz)
                    w = plsc.load_gather(
                        rows_v, [jnp.broadcast_to(p, (16,)), cb * 16 + l])
                    hits = ((lax.shift_right_logical(w, iota)) & 1) == 1
                    jv = ((cb * 16 + l) * 16 + iota).astype(jnp.float32)
                    pre = plsc.cumsum(jnp.where(hits, 1, 0))
                    keep = hits & ((cnt + pre) <= _MAXC)
                    plsc.store_compressed(
                        out_v.at[pl.ds(p * _MAXC + cnt, 16)], jv, mask=keep)
                    return (nz & (iota != l),
                            cnt + jnp.sum(jnp.where(keep, 1, 0)))

                nz, cnt = lax.while_loop(cond, body, (nz, cnt))
                return cnt

            lax.fori_loop(0, _NGRP // 16, per_wchunk, jnp.int32(0))
            return 0

        lax.fori_loop(0, _ROWS, per_row, 0)

        pltpu.sync_copy(ordf_v.at[pl.ds(row0, _ROWS)],
                        idxs_hbm.at[pl.ds(bbase + row0, _ROWS)])
        pltpu.sync_copy(out_v.at[pl.ds(0, _ROWS * _MAXC)],
                        nbr_hbm.at[pl.ds((bbase + row0) * _MAXC,
                                         _ROWS * _MAXC)])
        return 0

    lax.fori_loop(0, _NCHUNK, per_chunk, 0)


@jax.jit
def kernel(locs):
    xs = locs[..., 0].reshape(_B * _N)
    ys = locs[..., 1].reshape(_B * _N)
    zs = locs[..., 2].reshape(_B * _N)
    key = _compute_keys(xs, ys, zs)
    key_s, xs_s, ys_s, zs_s = _make_rank_scatter()(key, xs, ys, zs)
    locs_s = jnp.stack(
        [xs_s.reshape(_B, _N), ys_s.reshape(_B, _N), zs_s.reshape(_B, _N)],
        axis=-1)
    locs_sT = jnp.stack(
        [xs_s.reshape(_B, _N), ys_s.reshape(_B, _N), zs_s.reshape(_B, _N)],
        axis=1)
    bits = _compute_bits(locs_s, locs_sT).reshape(_B * _N, _NGRP)
    idxs, nbrs = _make_emit()(key_s, bits)
    return idxs.reshape(_B, _N), nbrs.reshape(_B, _N, _MAXC)
